# baseline (speedup 1.0000x reference)
import functools

import jax
import jax.numpy as jnp
from jax import lax
from jax.experimental import pallas as pl
from jax.experimental.pallas import tpu as pltpu

KS = 5
K2 = KS * KS
N = 50000
E = 800000
E_PAD = 802816
N_ACC = 50176
G = 50
GP = 64


def _pre_body(src_ref, dst_ref, px_ref, py_ref,
              g0, g1, g2, g3, w0, w1, w2, w3, dp):
    j = pl.program_id(0)
    eidx = j * 1024 + lax.broadcasted_iota(jnp.int32, (1, 1024), 1)
    valid = eidx < E
    src = src_ref[0]
    dst = dst_ref[0]
    vx = px_ref[0] * (KS - 1)
    vy = py_ref[0] * (KS - 1)
    i0x = jnp.clip(jnp.floor(vx), 0.0, KS - 2.0)
    i0y = jnp.clip(jnp.floor(vy), 0.0, KS - 2.0)
    fx = vx - i0x
    fy = vy - i0y
    ix = i0x.astype(jnp.int32)
    iy = i0y.astype(jnp.int32)
    base = src * K2
    zero_i = jnp.zeros_like(base)
    vmask = valid[0] if valid.ndim == 2 else valid
    g0[0] = jnp.where(vmask, base + ix + KS * iy, zero_i)
    g1[0] = jnp.where(vmask, base + ix + 1 + KS * iy, zero_i)
    g2[0] = jnp.where(vmask, base + ix + KS * (iy + 1), zero_i)
    g3[0] = jnp.where(vmask, base + ix + 1 + KS * (iy + 1), zero_i)
    zf = jnp.zeros_like(fx)
    w0[0] = jnp.where(vmask, (1.0 - fx) * (1.0 - fy), zf)
    w1[0] = jnp.where(vmask, fx * (1.0 - fy), zf)
    w2[0] = jnp.where(vmask, (1.0 - fx) * fy, zf)
    w3[0] = jnp.where(vmask, fx * fy, zf)
    dp[0] = jnp.where(vmask, dst, jnp.full_like(dst, N))


def _preprocess(src, dst, px, py):
    nb = E_PAD // 1024
    shp3 = (nb, 1, 1024)
    ospec = pl.BlockSpec((1, 1, 1024), lambda j: (j, 0, 0))
    out_shapes = ([jax.ShapeDtypeStruct(shp3, jnp.int32)] * 4
                  + [jax.ShapeDtypeStruct(shp3, jnp.float32)] * 4
                  + [jax.ShapeDtypeStruct(shp3, jnp.int32)])
    outs = pl.pallas_call(
        _pre_body,
        grid=(nb,),
        in_specs=[pl.BlockSpec((1, 1, 1024), lambda j: (j, 0, 0))] * 4,
        out_specs=[ospec] * 9,
        out_shape=out_shapes,
    )(src.reshape(shp3), dst.reshape(shp3),
      px.reshape(shp3), py.reshape(shp3))
    gs = [o.reshape(E_PAD) for o in outs[:4]]
    ws = [o.reshape(E_PAD) for o in outs[4:8]]
    dp = outs[8].reshape(E_PAD)
    return gs, ws, dp


def _ymm_body(x_ref, w_ref, o_ref):
    o_ref[0] = jnp.dot(x_ref[...], w_ref[0],
                       preferred_element_type=jnp.float32)


def _ymm(x, wt, bn=2000):
    fin = x.shape[1]
    kf = wt.shape[2]
    return pl.pallas_call(
        _ymm_body,
        grid=(2, N // bn),
        in_specs=[
            pl.BlockSpec((bn, fin), lambda c, i: (i, 0)),
            pl.BlockSpec((1, fin, kf), lambda c, i: (c, 0, 0)),
        ],
        out_specs=pl.BlockSpec((1, bn, kf), lambda c, i: (c, i, 0)),
        out_shape=jax.ShapeDtypeStruct((2, N, kf), jnp.float32),
    )(x, wt)


def _post_body(a_ref, d_ref, x_ref, r_ref, b_ref, o_ref):
    deg = d_ref[0, :, 0:1] + d_ref[1, :, 0:1]
    inv = 1.0 / jnp.maximum(deg, 1.0)
    xr = jnp.dot(x_ref[...], r_ref[...],
                 preferred_element_type=jnp.float32) + b_ref[...]
    a = jnp.concatenate([a_ref[0], a_ref[1]], axis=-1)
    z = a * inv + xr
    o_ref[...] = jnp.where(z > 0, z, jnp.exp(jnp.minimum(z, 0.0)) - 1.0)


def _post(agg, deg, x, root, b, bn=5000):
    fin = x.shape[1]
    f = agg.shape[2]
    return pl.pallas_call(
        _post_body,
        grid=(N // bn,),
        in_specs=[
            pl.BlockSpec((2, bn, f), lambda i: (0, i, 0)),
            pl.BlockSpec((2, bn, 16), lambda i: (0, i, 0)),
            pl.BlockSpec((bn, fin), lambda i: (i, 0)),
            pl.BlockSpec((fin, 2 * f), lambda i: (0, 0)),
            pl.BlockSpec((1, 2 * f), lambda i: (0, 0)),
        ],
        out_specs=pl.BlockSpec((bn, 2 * f), lambda i: (i, 0)),
        out_shape=jax.ShapeDtypeStruct((N, 2 * f), jnp.float32),
    )(agg, deg, x, root, b)


def _tail_body(h_ref, lo_ref, hi_ref, wf_ref, bf_ref, o_ref, s_ref, c_ref):
    k = pl.program_id(0)
    nk = pl.num_programs(0)
    bn = h_ref.shape[0]

    @pl.when(k == 0)
    def _init():
        s_ref[...] = jnp.zeros_like(s_ref)
        c_ref[...] = jnp.zeros_like(c_ref)

    gi = k * bn + lax.broadcasted_iota(jnp.int32, (GP, bn), 1)
    lo = lo_ref[...]
    hi = hi_ref[...]
    mask = ((gi >= lo) & (gi < hi)).astype(jnp.float32)
    s_ref[...] += jnp.dot(mask, h_ref[...],
                          preferred_element_type=jnp.float32)
    cnt = jnp.sum(mask, axis=1, keepdims=True)
    c_ref[...] += jnp.broadcast_to(cnt, c_ref.shape)

    @pl.when(k == nk - 1)
    def _fin():
        cnt_all = c_ref[:, 0:1]
        g = s_ref[...] / jnp.maximum(cnt_all, 1.0)
        logits = jnp.dot(g, wf_ref[...],
                         preferred_element_type=jnp.float32) + bf_ref[...]
        m = jnp.max(logits, axis=-1, keepdims=True)
        ex = jnp.exp(logits - m)
        lse = jnp.log(jnp.sum(ex, axis=-1, keepdims=True)) + m
        o_ref[...] = (logits - lse)[:G]


def _tail(h, lo, hi, wf, bf, bn=5000):
    fo = h.shape[1]
    no = wf.shape[1]
    return pl.pallas_call(
        _tail_body,
        grid=(N // bn,),
        in_specs=[
            pl.BlockSpec((bn, fo), lambda k: (k, 0)),
            pl.BlockSpec((GP, 1), lambda k: (0, 0)),
            pl.BlockSpec((GP, 1), lambda k: (0, 0)),
            pl.BlockSpec((fo, no), lambda k: (0, 0)),
            pl.BlockSpec((1, no), lambda k: (0, 0)),
        ],
        out_specs=pl.BlockSpec((G, no), lambda k: (0, 0)),
        out_shape=jax.ShapeDtypeStruct((G, no), jnp.float32),
        scratch_shapes=[
            pltpu.VMEM((GP, fo), jnp.float32),
            pltpu.VMEM((GP, 128), jnp.float32),
        ],
    )(h, lo, hi, wf, bf)


def _agg_placeholder(yv, gs, ws, dp, f):
    nr = N * K2
    outs = []
    for c in (0, 1):
        msg = (ws[0][:, None] * yv[gs[0] + c * nr]
               + ws[1][:, None] * yv[gs[1] + c * nr]
               + ws[2][:, None] * yv[gs[2] + c * nr]
               + ws[3][:, None] * yv[gs[3] + c * nr])
        acc = jnp.zeros((N_ACC, f), jnp.float32).at[dp].add(msg)
        outs.append(acc)
    return jnp.stack(outs)


def _deg_placeholder(dp):
    d = jnp.zeros((N_ACC, 16), jnp.float32).at[dp].add(
        jnp.ones((E_PAD, 16), jnp.float32))
    return jnp.stack([d, jnp.zeros_like(d)])


def _layer(h, gs, ws, dp, deg, W, root, b):
    fin = h.shape[1]
    fout = W.shape[2]
    f = fout // 2
    wt = jnp.stack([
        W[:, :, :f].transpose(1, 0, 2).reshape(fin, K2 * f),
        W[:, :, f:].transpose(1, 0, 2).reshape(fin, K2 * f),
    ])
    y = _ymm(h, wt)
    yv = y.reshape(2 * N * K2, f)
    agg = _agg_placeholder(yv, gs, ws, dp, f)
    return _post(agg, deg, h, root, b.reshape(1, fout))


def kernel(x, edge_index, pseudo, slice, W1, root1, b1, W2, root2, b2, Wf, bf):
    src = edge_index[0].astype(jnp.int32)
    dst = edge_index[1].astype(jnp.int32)
    pad = (0, E_PAD - E)
    srcp = jnp.pad(src, pad)
    dstp0 = jnp.pad(dst, pad)
    px = jnp.pad(pseudo[:, 0], pad)
    py = jnp.pad(pseudo[:, 1], pad)

    gs, ws, dp = _preprocess(srcp, dstp0, px, py)
    deg = _deg_placeholder(dp)

    h1 = _layer(x, gs, ws, dp, deg, W1, root1, b1)
    h2 = _layer(h1, gs, ws, dp, deg, W2, root2, b2)

    slc = slice.astype(jnp.int32)
    lo = jnp.full((GP,), N, jnp.int32).at[:G].set(slc[:G]).reshape(GP, 1)
    hi = jnp.full((GP,), N, jnp.int32).at[:G].set(slc[1:G + 1]).reshape(GP, 1)
    return _tail(h2, lo, hi, Wf, bf.reshape(1, -1))


# speedup vs baseline: 3.8266x; 3.8266x over previous
import functools

import jax
import jax.numpy as jnp
from jax import lax
from jax.experimental import pallas as pl
from jax.experimental.pallas import tpu as pltpu
from jax.experimental.pallas import tpu_sc as plsc

KS = 5
K2 = KS * KS
N = 50000
E = 800000
E_PAD = 802816
N_ACC = 50176
G = 50
GP = 64


def _pre_body(src_ref, dst_ref, px_ref, py_ref,
              g0, g1, g2, g3, w0, w1, w2, w3, dp):
    j = pl.program_id(0)
    eidx = j * 1024 + lax.broadcasted_iota(jnp.int32, (1, 1024), 1)
    valid = eidx < E
    src = src_ref[0]
    dst = dst_ref[0]
    vx = px_ref[0] * (KS - 1)
    vy = py_ref[0] * (KS - 1)
    i0x = jnp.clip(jnp.floor(vx), 0.0, KS - 2.0)
    i0y = jnp.clip(jnp.floor(vy), 0.0, KS - 2.0)
    fx = vx - i0x
    fy = vy - i0y
    ix = i0x.astype(jnp.int32)
    iy = i0y.astype(jnp.int32)
    base = src * K2
    zero_i = jnp.zeros_like(base)
    vmask = valid[0] if valid.ndim == 2 else valid
    g0[0] = jnp.where(vmask, base + ix + KS * iy, zero_i)
    g1[0] = jnp.where(vmask, base + ix + 1 + KS * iy, zero_i)
    g2[0] = jnp.where(vmask, base + ix + KS * (iy + 1), zero_i)
    g3[0] = jnp.where(vmask, base + ix + 1 + KS * (iy + 1), zero_i)
    zf = jnp.zeros_like(fx)
    w0[0] = jnp.where(vmask, (1.0 - fx) * (1.0 - fy), zf)
    w1[0] = jnp.where(vmask, fx * (1.0 - fy), zf)
    w2[0] = jnp.where(vmask, (1.0 - fx) * fy, zf)
    w3[0] = jnp.where(vmask, fx * fy, zf)
    dp[0] = jnp.where(vmask, dst, jnp.full_like(dst, N))


def _preprocess(src, dst, px, py):
    nb = E_PAD // 1024
    shp3 = (nb, 1, 1024)
    ospec = pl.BlockSpec((1, 1, 1024), lambda j: (j, 0, 0))
    out_shapes = ([jax.ShapeDtypeStruct(shp3, jnp.int32)] * 4
                  + [jax.ShapeDtypeStruct(shp3, jnp.float32)] * 4
                  + [jax.ShapeDtypeStruct(shp3, jnp.int32)])
    outs = pl.pallas_call(
        _pre_body,
        grid=(nb,),
        in_specs=[pl.BlockSpec((1, 1, 1024), lambda j: (j, 0, 0))] * 4,
        out_specs=[ospec] * 9,
        out_shape=out_shapes,
    )(src.reshape(shp3), dst.reshape(shp3),
      px.reshape(shp3), py.reshape(shp3))
    gs = [o.reshape(E_PAD) for o in outs[:4]]
    ws = [o.reshape(E_PAD) for o in outs[4:8]]
    dp = outs[8].reshape(E_PAD)
    return gs, ws, dp


def _ymm_body(x_ref, w_ref, o_ref):
    o_ref[0] = jnp.dot(x_ref[...], w_ref[0],
                       preferred_element_type=jnp.float32)


def _ymm(x, wt, bn=2000):
    fin = x.shape[1]
    kf = wt.shape[2]
    return pl.pallas_call(
        _ymm_body,
        grid=(2, N // bn),
        in_specs=[
            pl.BlockSpec((bn, fin), lambda c, i: (i, 0)),
            pl.BlockSpec((1, fin, kf), lambda c, i: (c, 0, 0)),
        ],
        out_specs=pl.BlockSpec((1, bn, kf), lambda c, i: (c, i, 0)),
        out_shape=jax.ShapeDtypeStruct((2, N, kf), jnp.float32),
    )(x, wt)


def _post_body(a_ref, d_ref, x_ref, r_ref, b_ref, o_ref):
    deg = d_ref[0, :, 0:1] + d_ref[1, :, 0:1]
    inv = 1.0 / jnp.maximum(deg, 1.0)
    xr = jnp.dot(x_ref[...], r_ref[...],
                 preferred_element_type=jnp.float32) + b_ref[...]
    a = jnp.concatenate([a_ref[0], a_ref[1]], axis=-1)
    z = a * inv + xr
    o_ref[...] = jnp.where(z > 0, z, jnp.exp(jnp.minimum(z, 0.0)) - 1.0)


def _post(agg, deg, x, root, b, bn=5000):
    fin = x.shape[1]
    f = agg.shape[2]
    return pl.pallas_call(
        _post_body,
        grid=(N // bn,),
        in_specs=[
            pl.BlockSpec((2, bn, f), lambda i: (0, i, 0)),
            pl.BlockSpec((2, bn, 16), lambda i: (0, i, 0)),
            pl.BlockSpec((bn, fin), lambda i: (i, 0)),
            pl.BlockSpec((fin, 2 * f), lambda i: (0, 0)),
            pl.BlockSpec((1, 2 * f), lambda i: (0, 0)),
        ],
        out_specs=pl.BlockSpec((bn, 2 * f), lambda i: (i, 0)),
        out_shape=jax.ShapeDtypeStruct((N, 2 * f), jnp.float32),
    )(agg, deg, x, root, b)


def _tail_body(h_ref, lo_ref, hi_ref, wf_ref, bf_ref, o_ref, s_ref, c_ref):
    k = pl.program_id(0)
    nk = pl.num_programs(0)
    bn = h_ref.shape[0]

    @pl.when(k == 0)
    def _init():
        s_ref[...] = jnp.zeros_like(s_ref)
        c_ref[...] = jnp.zeros_like(c_ref)

    gi = k * bn + lax.broadcasted_iota(jnp.int32, (GP, bn), 1)
    lo = lo_ref[...]
    hi = hi_ref[...]
    mask = ((gi >= lo) & (gi < hi)).astype(jnp.float32)
    s_ref[...] += jnp.dot(mask, h_ref[...],
                          preferred_element_type=jnp.float32)
    cnt = jnp.sum(mask, axis=1, keepdims=True)
    c_ref[...] += jnp.broadcast_to(cnt, c_ref.shape)

    @pl.when(k == nk - 1)
    def _fin():
        cnt_all = c_ref[:, 0:1]
        g = s_ref[...] / jnp.maximum(cnt_all, 1.0)
        logits = jnp.dot(g, wf_ref[...],
                         preferred_element_type=jnp.float32) + bf_ref[...]
        m = jnp.max(logits, axis=-1, keepdims=True)
        ex = jnp.exp(logits - m)
        lse = jnp.log(jnp.sum(ex, axis=-1, keepdims=True)) + m
        o_ref[...] = (logits - lse)[:G]


def _tail(h, lo, hi, wf, bf, bn=5000):
    fo = h.shape[1]
    no = wf.shape[1]
    return pl.pallas_call(
        _tail_body,
        grid=(N // bn,),
        in_specs=[
            pl.BlockSpec((bn, fo), lambda k: (k, 0)),
            pl.BlockSpec((GP, 1), lambda k: (0, 0)),
            pl.BlockSpec((GP, 1), lambda k: (0, 0)),
            pl.BlockSpec((fo, no), lambda k: (0, 0)),
            pl.BlockSpec((1, no), lambda k: (0, 0)),
        ],
        out_specs=pl.BlockSpec((G, no), lambda k: (0, 0)),
        out_shape=jax.ShapeDtypeStruct((G, no), jnp.float32),
        scratch_shapes=[
            pltpu.VMEM((GP, fo), jnp.float32),
            pltpu.VMEM((GP, 128), jnp.float32),
        ],
    )(h, lo, hi, wf, bf)


_B = 128
_EPT = E_PAD // 16
_NBLK = _EPT // _B
_RPT = N_ACC // 16
_ZR = 56


def _sc_mesh():
    return plsc.VectorSubcoreMesh(core_axis_name="c", subcore_axis_name="s")


def _agg_sc(yv, gs, ws, dp, f):
    nrow = N * K2

    @functools.partial(
        pl.kernel,
        mesh=_sc_mesh(),
        compiler_params=pltpu.CompilerParams(use_tc_tiling_on_sc=False),
        out_type=jax.ShapeDtypeStruct((2 * N_ACC, f), jnp.float32),
        scratch_types=[
            pltpu.VMEM((_B,), jnp.int32),
            pltpu.VMEM((_B,), jnp.int32),
            pltpu.VMEM((_B,), jnp.int32),
            pltpu.VMEM((_B,), jnp.int32),
            pltpu.VMEM((_B,), jnp.int32),
            pltpu.VMEM((_B,), jnp.float32),
            pltpu.VMEM((_B,), jnp.float32),
            pltpu.VMEM((_B,), jnp.float32),
            pltpu.VMEM((_B,), jnp.float32),
            pltpu.VMEM((_B, f), jnp.float32),
            pltpu.VMEM((_B, f), jnp.float32),
            pltpu.VMEM((_B, f), jnp.float32),
            pltpu.VMEM((_B, f), jnp.float32),
            pltpu.VMEM((_B, f), jnp.float32),
            pltpu.VMEM((_ZR, f), jnp.float32),
            pltpu.VMEM_SHARED((N_ACC, f), jnp.float32),
            pltpu.SemaphoreType.DMA,
            pltpu.SemaphoreType.DMA,
            pltpu.SemaphoreType.DMA,
            pltpu.SemaphoreType.DMA,
        ],
    )
    def k(g0h, g1h, g2h, g3h, w0h, w1h, w2h, w3h, dph, yvh, acch,
          d_v, g0v, g1v, g2v, g3v, w0v, w1v, w2v, w3v,
          r0, r1, r2, r3, m_v, zb, acc_sh, s0, s1, s2, s3):
        cid = lax.axis_index("c")
        sid = lax.axis_index("s")

        def zrow(i, _):
            for j in range(f // 16):
                zb[i, pl.ds(j * 16, 16)] = jnp.zeros((16,), jnp.float32)
            return 0
        lax.fori_loop(0, _ZR, zrow, 0)

        def zcp(i, _):
            pltpu.sync_copy(zb, acc_sh.at[pl.ds(sid * _RPT + i * _ZR, _ZR)])
            return 0
        lax.fori_loop(0, _RPT // _ZR, zcp, 0)
        plsc.subcore_barrier()

        off0 = cid * nrow
        base = sid * _EPT

        def blk(bi, _):
            off = base + bi * _B
            pltpu.sync_copy(dph.at[pl.ds(off, _B)], d_v)
            pltpu.sync_copy(g0h.at[pl.ds(off, _B)], g0v)
            pltpu.sync_copy(g1h.at[pl.ds(off, _B)], g1v)
            pltpu.sync_copy(g2h.at[pl.ds(off, _B)], g2v)
            pltpu.sync_copy(g3h.at[pl.ds(off, _B)], g3v)
            pltpu.sync_copy(w0h.at[pl.ds(off, _B)], w0v)
            pltpu.sync_copy(w1h.at[pl.ds(off, _B)], w1v)
            pltpu.sync_copy(w2h.at[pl.ds(off, _B)], w2v)
            pltpu.sync_copy(w3h.at[pl.ds(off, _B)], w3v)

            def addo(j, _):
                sl = pl.ds(j * 16, 16)
                g0v[sl] = g0v[sl] + off0
                g1v[sl] = g1v[sl] + off0
                g2v[sl] = g2v[sl] + off0
                g3v[sl] = g3v[sl] + off0
                return 0
            lax.fori_loop(0, _B // 16, addo, 0)

            c0 = pltpu.async_copy(yvh.at[g0v], r0, s0)
            c1 = pltpu.async_copy(yvh.at[g1v], r1, s1)
            c2 = pltpu.async_copy(yvh.at[g2v], r2, s2)
            c3 = pltpu.async_copy(yvh.at[g3v], r3, s3)
            c0.wait()
            c1.wait()
            c2.wait()
            c3.wait()

            def ed16(g, _):
                gsl = pl.ds(g * 16, 16)
                vb0 = w0v[gsl]
                vb1 = w1v[gsl]
                vb2 = w2v[gsl]
                vb3 = w3v[gsl]
                for t in range(16):
                    b = g * 16 + t
                    for j in range(f // 16):
                        sl = pl.ds(j * 16, 16)
                        m_v[b, sl] = (r0[b, sl] * vb0[t] + r1[b, sl] * vb1[t]
                                      + r2[b, sl] * vb2[t] + r3[b, sl] * vb3[t])
                return 0
            lax.fori_loop(0, _B // 16, ed16, 0)

            pltpu.sync_copy(m_v, acc_sh.at[d_v], add=True)
            return 0
        lax.fori_loop(0, _NBLK, blk, 0)
        plsc.subcore_barrier()

        def wb(i, _):
            r0b = sid * _RPT + i * _ZR
            pltpu.sync_copy(acc_sh.at[pl.ds(r0b, _ZR)], zb)
            pltpu.sync_copy(zb, acch.at[pl.ds(cid * N_ACC + r0b, _ZR)])
            return 0
        lax.fori_loop(0, _RPT // _ZR, wb, 0)

    out = k(gs[0], gs[1], gs[2], gs[3], ws[0], ws[1], ws[2], ws[3], dp, yv)
    return out.reshape(2, N_ACC, f)


def _deg_sc(dp):
    ept = E_PAD // 32
    nblk = ept // _B

    @functools.partial(
        pl.kernel,
        mesh=_sc_mesh(),
        compiler_params=pltpu.CompilerParams(use_tc_tiling_on_sc=False),
        out_type=jax.ShapeDtypeStruct((2 * N_ACC, 16), jnp.float32),
        scratch_types=[
            pltpu.VMEM((_B,), jnp.int32),
            pltpu.VMEM((_B, 16), jnp.float32),
            pltpu.VMEM((_ZR, 16), jnp.float32),
            pltpu.VMEM_SHARED((N_ACC, 16), jnp.float32),
        ],
    )
    def k(dph, acch, d_v, ones_v, zb, acc_sh):
        cid = lax.axis_index("c")
        sid = lax.axis_index("s")

        def zrow(i, _):
            zb[i, :] = jnp.zeros((16,), jnp.float32)
            return 0
        lax.fori_loop(0, _ZR, zrow, 0)

        def orow(i, _):
            ones_v[i, :] = jnp.ones((16,), jnp.float32)
            return 0
        lax.fori_loop(0, _B, orow, 0)

        def zcp(i, _):
            pltpu.sync_copy(zb, acc_sh.at[pl.ds(sid * _RPT + i * _ZR, _ZR)])
            return 0
        lax.fori_loop(0, _RPT // _ZR, zcp, 0)
        plsc.subcore_barrier()

        base = (sid * 2 + cid) * ept

        def blk(bi, _):
            pltpu.sync_copy(dph.at[pl.ds(base + bi * _B, _B)], d_v)
            pltpu.sync_copy(ones_v, acc_sh.at[d_v], add=True)
            return 0
        lax.fori_loop(0, nblk, blk, 0)
        plsc.subcore_barrier()

        def wb(i, _):
            r0b = sid * _RPT + i * _ZR
            pltpu.sync_copy(acc_sh.at[pl.ds(r0b, _ZR)], zb)
            pltpu.sync_copy(zb, acch.at[pl.ds(cid * N_ACC + r0b, _ZR)])
            return 0
        lax.fori_loop(0, _RPT // _ZR, wb, 0)

    return k(dp).reshape(2, N_ACC, 16)


def _layer(h, gs, ws, dp, deg, W, root, b):
    fin = h.shape[1]
    fout = W.shape[2]
    f = fout // 2
    wt = jnp.stack([
        W[:, :, :f].transpose(1, 0, 2).reshape(fin, K2 * f),
        W[:, :, f:].transpose(1, 0, 2).reshape(fin, K2 * f),
    ])
    y = _ymm(h, wt)
    yv = y.reshape(2 * N * K2, f)
    agg = _agg_sc(yv, gs, ws, dp, f)
    return _post(agg, deg, h, root, b.reshape(1, fout))


def kernel(x, edge_index, pseudo, slice, W1, root1, b1, W2, root2, b2, Wf, bf):
    src = edge_index[0].astype(jnp.int32)
    dst = edge_index[1].astype(jnp.int32)
    pad = (0, E_PAD - E)
    srcp = jnp.pad(src, pad)
    dstp0 = jnp.pad(dst, pad)
    px = jnp.pad(pseudo[:, 0], pad)
    py = jnp.pad(pseudo[:, 1], pad)

    gs, ws, dp = _preprocess(srcp, dstp0, px, py)
    deg = _deg_sc(dp)

    h1 = _layer(x, gs, ws, dp, deg, W1, root1, b1)
    h2 = _layer(h1, gs, ws, dp, deg, W2, root2, b2)

    slc = slice.astype(jnp.int32)
    lo = jnp.full((GP,), N, jnp.int32).at[:G].set(slc[:G]).reshape(GP, 1)
    hi = jnp.full((GP,), N, jnp.int32).at[:G].set(slc[1:G + 1]).reshape(GP, 1)
    return _tail(h2, lo, hi, Wf, bf.reshape(1, -1))


# speedup vs baseline: 6.0595x; 1.5835x over previous
import functools

import jax
import jax.numpy as jnp
from jax import lax
from jax.experimental import pallas as pl
from jax.experimental.pallas import tpu as pltpu
from jax.experimental.pallas import tpu_sc as plsc

KS = 5
K2 = KS * KS
N = 50000
E = 800000
E_PAD = 802816
N_ACC = 50176
G = 50
GP = 64


def _pre_body(src_ref, dst_ref, px_ref, py_ref,
              g0, g1, g2, g3, w0, w1, w2, w3, dp):
    j = pl.program_id(0)
    eidx = j * 1024 + lax.broadcasted_iota(jnp.int32, (1, 1024), 1)
    valid = eidx < E
    src = src_ref[0]
    dst = dst_ref[0]
    vx = px_ref[0] * (KS - 1)
    vy = py_ref[0] * (KS - 1)
    i0x = jnp.clip(jnp.floor(vx), 0.0, KS - 2.0)
    i0y = jnp.clip(jnp.floor(vy), 0.0, KS - 2.0)
    fx = vx - i0x
    fy = vy - i0y
    ix = i0x.astype(jnp.int32)
    iy = i0y.astype(jnp.int32)
    base = src * K2
    zero_i = jnp.zeros_like(base)
    vmask = valid[0] if valid.ndim == 2 else valid
    g0[0] = jnp.where(vmask, base + ix + KS * iy, zero_i)
    g1[0] = jnp.where(vmask, base + ix + 1 + KS * iy, zero_i)
    g2[0] = jnp.where(vmask, base + ix + KS * (iy + 1), zero_i)
    g3[0] = jnp.where(vmask, base + ix + 1 + KS * (iy + 1), zero_i)
    zf = jnp.zeros_like(fx)
    w0[0] = jnp.where(vmask, (1.0 - fx) * (1.0 - fy), zf)
    w1[0] = jnp.where(vmask, fx * (1.0 - fy), zf)
    w2[0] = jnp.where(vmask, (1.0 - fx) * fy, zf)
    w3[0] = jnp.where(vmask, fx * fy, zf)
    dp[0] = jnp.where(vmask, dst, jnp.full_like(dst, N))


def _preprocess(src, dst, px, py):
    nb = E_PAD // 1024
    shp3 = (nb, 1, 1024)
    ospec = pl.BlockSpec((1, 1, 1024), lambda j: (j, 0, 0))
    out_shapes = ([jax.ShapeDtypeStruct(shp3, jnp.int32)] * 4
                  + [jax.ShapeDtypeStruct(shp3, jnp.float32)] * 4
                  + [jax.ShapeDtypeStruct(shp3, jnp.int32)])
    outs = pl.pallas_call(
        _pre_body,
        grid=(nb,),
        in_specs=[pl.BlockSpec((1, 1, 1024), lambda j: (j, 0, 0))] * 4,
        out_specs=[ospec] * 9,
        out_shape=out_shapes,
    )(src.reshape(shp3), dst.reshape(shp3),
      px.reshape(shp3), py.reshape(shp3))
    gs = [o.reshape(E_PAD) for o in outs[:4]]
    ws = [o.reshape(E_PAD) for o in outs[4:8]]
    dp = outs[8].reshape(E_PAD)
    return gs, ws, dp


def _ymm_body(x_ref, w_ref, o_ref):
    o_ref[0] = jnp.dot(x_ref[...], w_ref[0],
                       preferred_element_type=jnp.float32)


def _ymm(x, wt, bn=2000):
    fin = x.shape[1]
    kf = wt.shape[2]
    return pl.pallas_call(
        _ymm_body,
        grid=(2, N // bn),
        in_specs=[
            pl.BlockSpec((bn, fin), lambda c, i: (i, 0)),
            pl.BlockSpec((1, fin, kf), lambda c, i: (c, 0, 0)),
        ],
        out_specs=pl.BlockSpec((1, bn, kf), lambda c, i: (c, i, 0)),
        out_shape=jax.ShapeDtypeStruct((2, N, kf), jnp.float32),
    )(x, wt)


def _post_body(a_ref, d_ref, x_ref, r_ref, b_ref, o_ref):
    deg = d_ref[0, :, 0:1] + d_ref[1, :, 0:1]
    inv = 1.0 / jnp.maximum(deg, 1.0)
    xr = jnp.dot(x_ref[...], r_ref[...],
                 preferred_element_type=jnp.float32) + b_ref[...]
    a = jnp.concatenate([a_ref[0], a_ref[1]], axis=-1)
    z = a * inv + xr
    o_ref[...] = jnp.where(z > 0, z, jnp.exp(jnp.minimum(z, 0.0)) - 1.0)


def _post(agg, deg, x, root, b, bn=5000):
    fin = x.shape[1]
    f = agg.shape[2]
    return pl.pallas_call(
        _post_body,
        grid=(N // bn,),
        in_specs=[
            pl.BlockSpec((2, bn, f), lambda i: (0, i, 0)),
            pl.BlockSpec((2, bn, 16), lambda i: (0, i, 0)),
            pl.BlockSpec((bn, fin), lambda i: (i, 0)),
            pl.BlockSpec((fin, 2 * f), lambda i: (0, 0)),
            pl.BlockSpec((1, 2 * f), lambda i: (0, 0)),
        ],
        out_specs=pl.BlockSpec((bn, 2 * f), lambda i: (i, 0)),
        out_shape=jax.ShapeDtypeStruct((N, 2 * f), jnp.float32),
    )(agg, deg, x, root, b)


def _tail_body(h_ref, lo_ref, hi_ref, wf_ref, bf_ref, o_ref, s_ref, c_ref):
    k = pl.program_id(0)
    nk = pl.num_programs(0)
    bn = h_ref.shape[0]

    @pl.when(k == 0)
    def _init():
        s_ref[...] = jnp.zeros_like(s_ref)
        c_ref[...] = jnp.zeros_like(c_ref)

    gi = k * bn + lax.broadcasted_iota(jnp.int32, (GP, bn), 1)
    lo = lo_ref[...]
    hi = hi_ref[...]
    mask = ((gi >= lo) & (gi < hi)).astype(jnp.float32)
    s_ref[...] += jnp.dot(mask, h_ref[...],
                          preferred_element_type=jnp.float32)
    cnt = jnp.sum(mask, axis=1, keepdims=True)
    c_ref[...] += jnp.broadcast_to(cnt, c_ref.shape)

    @pl.when(k == nk - 1)
    def _fin():
        cnt_all = c_ref[:, 0:1]
        g = s_ref[...] / jnp.maximum(cnt_all, 1.0)
        logits = jnp.dot(g, wf_ref[...],
                         preferred_element_type=jnp.float32) + bf_ref[...]
        m = jnp.max(logits, axis=-1, keepdims=True)
        ex = jnp.exp(logits - m)
        lse = jnp.log(jnp.sum(ex, axis=-1, keepdims=True)) + m
        o_ref[...] = (logits - lse)[:G]


def _tail(h, lo, hi, wf, bf, bn=5000):
    fo = h.shape[1]
    no = wf.shape[1]
    return pl.pallas_call(
        _tail_body,
        grid=(N // bn,),
        in_specs=[
            pl.BlockSpec((bn, fo), lambda k: (k, 0)),
            pl.BlockSpec((GP, 1), lambda k: (0, 0)),
            pl.BlockSpec((GP, 1), lambda k: (0, 0)),
            pl.BlockSpec((fo, no), lambda k: (0, 0)),
            pl.BlockSpec((1, no), lambda k: (0, 0)),
        ],
        out_specs=pl.BlockSpec((G, no), lambda k: (0, 0)),
        out_shape=jax.ShapeDtypeStruct((G, no), jnp.float32),
        scratch_shapes=[
            pltpu.VMEM((GP, fo), jnp.float32),
            pltpu.VMEM((GP, 128), jnp.float32),
        ],
    )(h, lo, hi, wf, bf)


_B = 128
_EPT = E_PAD // 16
_NBLK = _EPT // _B
_RPT = N_ACC // 16
_ZR = 56


def _sc_mesh():
    return plsc.VectorSubcoreMesh(core_axis_name="c", subcore_axis_name="s")


def _agg_sc(yv, gs, ws, dp, f):
    nrow = N * K2

    @functools.partial(
        pl.kernel,
        mesh=_sc_mesh(),
        compiler_params=pltpu.CompilerParams(use_tc_tiling_on_sc=False),
        out_type=jax.ShapeDtypeStruct((2 * N_ACC, f), jnp.float32),
        scratch_types=[
            pltpu.VMEM((4, _B), jnp.int32),
            pltpu.VMEM((4, _B), jnp.int32),
            pltpu.VMEM((4, _B), jnp.int32),
            pltpu.VMEM((4, _B), jnp.int32),
            pltpu.VMEM((4, _B), jnp.int32),
            pltpu.VMEM((4, _B), jnp.float32),
            pltpu.VMEM((4, _B), jnp.float32),
            pltpu.VMEM((4, _B), jnp.float32),
            pltpu.VMEM((4, _B), jnp.float32),
            pltpu.VMEM((_B, f), jnp.float32),
            pltpu.VMEM((_B, f), jnp.float32),
            pltpu.VMEM((_B, f), jnp.float32),
            pltpu.VMEM((_B, f), jnp.float32),
            pltpu.VMEM((_B, f), jnp.float32),
            pltpu.VMEM((_ZR, f), jnp.float32),
            pltpu.VMEM_SHARED((N_ACC, f), jnp.float32),
            pltpu.SemaphoreType.DMA,
            pltpu.SemaphoreType.DMA,
            pltpu.SemaphoreType.DMA,
            pltpu.SemaphoreType.DMA,
        ],
    )
    def k(g0h, g1h, g2h, g3h, w0h, w1h, w2h, w3h, dph, yvh, acch,
          d_v, g0v, g1v, g2v, g3v, w0v, w1v, w2v, w3v,
          r0, r1, r2, r3, m_v, zb, acc_sh, s0, s1, s2, s3):
        cid = lax.axis_index("c")
        sid = lax.axis_index("s")

        def zrow(i, _):
            for j in range(f // 16):
                zb[i, pl.ds(j * 16, 16)] = jnp.zeros((16,), jnp.float32)
            return 0
        lax.fori_loop(0, _ZR, zrow, 0)

        def zcp(i, _):
            pltpu.sync_copy(zb, acc_sh.at[pl.ds(sid * _RPT + i * _ZR, _ZR)])
            return 0
        lax.fori_loop(0, _RPT // _ZR, zcp, 0)
        plsc.subcore_barrier()

        off0 = cid * nrow
        rbase = sid * (_EPT // _B)

        def blk(ci, _):
            roff = rbase + ci * 4
            pltpu.sync_copy(dph.at[pl.ds(roff, 4)], d_v)
            pltpu.sync_copy(g0h.at[pl.ds(roff, 4)], g0v)
            pltpu.sync_copy(g1h.at[pl.ds(roff, 4)], g1v)
            pltpu.sync_copy(g2h.at[pl.ds(roff, 4)], g2v)
            pltpu.sync_copy(g3h.at[pl.ds(roff, 4)], g3v)
            pltpu.sync_copy(w0h.at[pl.ds(roff, 4)], w0v)
            pltpu.sync_copy(w1h.at[pl.ds(roff, 4)], w1v)
            pltpu.sync_copy(w2h.at[pl.ds(roff, 4)], w2v)
            pltpu.sync_copy(w3h.at[pl.ds(roff, 4)], w3v)

            for sb in range(4):
                def addo(j, _):
                    sl = pl.ds(j * 16, 16)
                    g0v[sb, sl] = g0v[sb, sl] + off0
                    g1v[sb, sl] = g1v[sb, sl] + off0
                    g2v[sb, sl] = g2v[sb, sl] + off0
                    g3v[sb, sl] = g3v[sb, sl] + off0
                    return 0
                lax.fori_loop(0, _B // 16, addo, 0)

                c0 = pltpu.async_copy(yvh.at[g0v.at[sb]], r0, s0)
                c1 = pltpu.async_copy(yvh.at[g1v.at[sb]], r1, s1)
                c2 = pltpu.async_copy(yvh.at[g2v.at[sb]], r2, s2)
                c3 = pltpu.async_copy(yvh.at[g3v.at[sb]], r3, s3)
                c0.wait()
                c1.wait()
                c2.wait()
                c3.wait()

                def ed16(g, _):
                    gsl = pl.ds(g * 16, 16)
                    vb0 = w0v[sb, gsl]
                    vb1 = w1v[sb, gsl]
                    vb2 = w2v[sb, gsl]
                    vb3 = w3v[sb, gsl]
                    for t in range(16):
                        b = g * 16 + t
                        for j in range(f // 16):
                            sl = pl.ds(j * 16, 16)
                            m_v[b, sl] = (r0[b, sl] * vb0[t]
                                          + r1[b, sl] * vb1[t]
                                          + r2[b, sl] * vb2[t]
                                          + r3[b, sl] * vb3[t])
                    return 0
                lax.fori_loop(0, _B // 16, ed16, 0)

                pltpu.sync_copy(m_v, acc_sh.at[d_v.at[sb]], add=True)
            return 0
        lax.fori_loop(0, _NBLK // 4, blk, 0)
        plsc.subcore_barrier()

        def wb(i, _):
            r0b = sid * _RPT + i * _ZR
            pltpu.sync_copy(acc_sh.at[pl.ds(r0b, _ZR)], zb)
            pltpu.sync_copy(zb, acch.at[pl.ds(cid * N_ACC + r0b, _ZR)])
            return 0
        lax.fori_loop(0, _RPT // _ZR, wb, 0)

    g2d = [g.reshape(E_PAD // _B, _B) for g in gs]
    w2d = [w.reshape(E_PAD // _B, _B) for w in ws]
    dp2d = dp.reshape(E_PAD // _B, _B)
    out = k(g2d[0], g2d[1], g2d[2], g2d[3],
            w2d[0], w2d[1], w2d[2], w2d[3], dp2d, yv)
    return out.reshape(2, N_ACC, f)


def _deg_sc(dp):
    ept = E_PAD // 32
    nblk = ept // _B

    @functools.partial(
        pl.kernel,
        mesh=_sc_mesh(),
        compiler_params=pltpu.CompilerParams(use_tc_tiling_on_sc=False),
        out_type=jax.ShapeDtypeStruct((2 * N_ACC, 16), jnp.float32),
        scratch_types=[
            pltpu.VMEM((4, _B), jnp.int32),
            pltpu.VMEM((_B, 16), jnp.float32),
            pltpu.VMEM((_ZR, 16), jnp.float32),
            pltpu.VMEM_SHARED((N_ACC, 16), jnp.float32),
        ],
    )
    def k(dph, acch, d_v, ones_v, zb, acc_sh):
        cid = lax.axis_index("c")
        sid = lax.axis_index("s")

        def zrow(i, _):
            zb[i, :] = jnp.zeros((16,), jnp.float32)
            return 0
        lax.fori_loop(0, _ZR, zrow, 0)

        def orow(i, _):
            ones_v[i, :] = jnp.ones((16,), jnp.float32)
            return 0
        lax.fori_loop(0, _B, orow, 0)

        def zcp(i, _):
            pltpu.sync_copy(zb, acc_sh.at[pl.ds(sid * _RPT + i * _ZR, _ZR)])
            return 0
        lax.fori_loop(0, _RPT // _ZR, zcp, 0)
        plsc.subcore_barrier()

        rbase = (sid * 2 + cid) * (ept // _B)

        def blk(ci, _):
            pltpu.sync_copy(dph.at[pl.ds(rbase + ci * 4, 4)], d_v)
            for sb in range(4):
                pltpu.sync_copy(ones_v, acc_sh.at[d_v.at[sb]], add=True)
            return 0
        lax.fori_loop(0, nblk // 4, blk, 0)
        plsc.subcore_barrier()

        def wb(i, _):
            r0b = sid * _RPT + i * _ZR
            pltpu.sync_copy(acc_sh.at[pl.ds(r0b, _ZR)], zb)
            pltpu.sync_copy(zb, acch.at[pl.ds(cid * N_ACC + r0b, _ZR)])
            return 0
        lax.fori_loop(0, _RPT // _ZR, wb, 0)

    return k(dp.reshape(E_PAD // _B, _B)).reshape(2, N_ACC, 16)


def _layer(h, gs, ws, dp, deg, W, root, b):
    fin = h.shape[1]
    fout = W.shape[2]
    f = fout // 2
    wt = jnp.stack([
        W[:, :, :f].transpose(1, 0, 2).reshape(fin, K2 * f),
        W[:, :, f:].transpose(1, 0, 2).reshape(fin, K2 * f),
    ])
    y = _ymm(h, wt)
    yv = y.reshape(2 * N * K2, f)
    agg = _agg_sc(yv, gs, ws, dp, f)
    return _post(agg, deg, h, root, b.reshape(1, fout))


def kernel(x, edge_index, pseudo, slice, W1, root1, b1, W2, root2, b2, Wf, bf):
    src = edge_index[0].astype(jnp.int32)
    dst = edge_index[1].astype(jnp.int32)
    pad = (0, E_PAD - E)
    srcp = jnp.pad(src, pad)
    dstp0 = jnp.pad(dst, pad)
    px = jnp.pad(pseudo[:, 0], pad)
    py = jnp.pad(pseudo[:, 1], pad)

    gs, ws, dp = _preprocess(srcp, dstp0, px, py)
    deg = _deg_sc(dp)

    h1 = _layer(x, gs, ws, dp, deg, W1, root1, b1)
    h2 = _layer(h1, gs, ws, dp, deg, W2, root2, b2)

    slc = slice.astype(jnp.int32)
    lo = jnp.full((GP,), N, jnp.int32).at[:G].set(slc[:G]).reshape(GP, 1)
    hi = jnp.full((GP,), N, jnp.int32).at[:G].set(slc[1:G + 1]).reshape(GP, 1)
    return _tail(h2, lo, hi, Wf, bf.reshape(1, -1))
